# Initial kernel scaffold; baseline (speedup 1.0000x reference)
import jax
import jax.numpy as jnp
from jax.experimental import pallas as pl


def kernel(x, pos, edge_attr, params, z, edge_index):
    raise NotImplementedError("write your pallas kernel here")


# speedup vs baseline: 1.9160x; 1.9160x over previous
import functools

import jax
import jax.numpy as jnp
from jax import lax
from jax.experimental import pallas as pl
from jax.experimental.pallas import tpu as pltpu
from jax.experimental.pallas import tpu_sc as plsc

N = 10000
E = 160000
HID = 128
HEADS = 4
HD = 32

NC = 2
NS = 16
NW = NC * NS
CH = 128
EPAD = 163840
NPAD = 10240

EB = 512
NB = 400


def _sc_gather(table, idx):
    bp = idx.shape[0]
    d = table.shape[1]
    bw = bp // NW
    nit = bw // CH
    mesh = plsc.VectorSubcoreMesh(core_axis_name="c", subcore_axis_name="s")

    @functools.partial(
        pl.kernel, mesh=mesh,
        out_type=jax.ShapeDtypeStruct((bp, d), jnp.float32),
        scratch_types=[
            pltpu.VMEM((CH,), jnp.int32),
            pltpu.VMEM((CH, d), jnp.float32),
            pltpu.SemaphoreType.DMA,
        ],
    )
    def k(table_hbm, idx_hbm, out_hbm, idx_v, rows_v, sem):
        cid = lax.axis_index("c")
        sid = lax.axis_index("s")
        base = (sid * NC + cid) * bw

        def body(j, c):
            off = base + j * CH
            pltpu.sync_copy(idx_hbm.at[pl.ds(off, CH)], idx_v)
            pltpu.async_copy(table_hbm.at[idx_v], rows_v, sem).wait()
            pltpu.sync_copy(rows_v, out_hbm.at[pl.ds(off, CH)])
            return c

        lax.fori_loop(0, nit, body, 0)

    return k(table, idx)


def _sc_scatter_add(vals, idx, zeros):
    d = vals.shape[1]
    bw = EPAD // NW
    nit = bw // CH
    rps = NPAD // NS
    mesh = plsc.VectorSubcoreMesh(core_axis_name="c", subcore_axis_name="s")

    @functools.partial(
        pl.kernel, mesh=mesh,
        out_type=jax.ShapeDtypeStruct((NC, NPAD, d), jnp.float32),
        scratch_types=[
            pltpu.VMEM((CH,), jnp.int32),
            pltpu.VMEM((CH, d), jnp.float32),
            pltpu.VMEM_SHARED((NPAD, d), jnp.float32),
        ],
    )
    def k(vals_hbm, idx_hbm, zeros_hbm, out_hbm, idx_v, val_v, acc_sh):
        cid = lax.axis_index("c")
        sid = lax.axis_index("s")

        @pl.when(sid == 0)
        def _():
            pltpu.sync_copy(zeros_hbm, acc_sh)

        plsc.subcore_barrier()
        base = (sid * NC + cid) * bw

        def body(j, c):
            off = base + j * CH
            pltpu.sync_copy(idx_hbm.at[pl.ds(off, CH)], idx_v)
            pltpu.sync_copy(vals_hbm.at[pl.ds(off, CH)], val_v)
            pltpu.sync_copy(val_v, acc_sh.at[idx_v], add=True)
            return c

        lax.fori_loop(0, nit, body, 0)
        plsc.subcore_barrier()
        pltpu.sync_copy(acc_sh.at[pl.ds(sid * rps, rps)],
                        out_hbm.at[cid, pl.ds(sid * rps, rps)])

    return k(vals, idx, zeros)


def _full(shape):
    return pl.BlockSpec(shape, lambda i: tuple(0 for _ in shape))


def _rows(b, w):
    return pl.BlockSpec((b, w), lambda i: (i, 0))


def _tc_h0(x, z2, w1, b1, w2, b2, embp):
    def body(x_r, z_r, w1_r, b1_r, w2_r, b2_r, emb_r, h0_o, ez_o):
        a = jnp.maximum(jnp.dot(x_r[...], w1_r[...],
                                preferred_element_type=jnp.float32)
                        + b1_r[...], 0.0)
        h0_o[...] = jnp.dot(a, w2_r[...],
                            preferred_element_type=jnp.float32) + b2_r[...]
        zc = z_r[...]
        oh = (zc == lax.broadcasted_iota(jnp.int32, (NB, 96), 1)
              ).astype(jnp.float32)
        ez_o[...] = jnp.dot(oh, emb_r[...],
                            preferred_element_type=jnp.float32)

    return pl.pallas_call(
        body,
        grid=(N // NB,),
        in_specs=[_rows(NB, 128), _rows(NB, 1), _full((128, 256)),
                  _full((1, 256)), _full((256, 256)), _full((1, 256)),
                  _full((96, 32))],
        out_specs=[_rows(NB, 256), _rows(NB, 32)],
        out_shape=[jax.ShapeDtypeStruct((N, 256), jnp.float32),
                   jax.ShapeDtypeStruct((N, 32), jnp.float32)],
    )(x, z2, w1, b1, w2, b2, embp)


def _tc_rf8(ps, pd, ea8):
    def body(ps_r, pd_r, ea_r, o_r):
        dlt = pd_r[:, 0:3] - ps_r[:, 0:3]
        r = jnp.sqrt(jnp.sum(dlt * dlt, axis=1, keepdims=True) + 1e-8)
        o_r[...] = jnp.concatenate(
            [r, ea_r[:, 0:4], jnp.zeros((EB, 3), jnp.float32)], axis=1)

    return pl.pallas_call(
        body,
        grid=(EPAD // EB,),
        in_specs=[_rows(EB, 128), _rows(EB, 128), _rows(EB, 8)],
        out_specs=_rows(EB, 8),
        out_shape=jax.ShapeDtypeStruct((EPAD, 8), jnp.float32),
    )(ps, pd, ea8)


def _tc_qkv(h, wq, wk, wv):
    din = h.shape[1]

    def body(h_r, wq_r, wk_r, wv_r, q_o, kv_o):
        hh = h_r[...]
        q_o[...] = jnp.dot(hh, wq_r[...], preferred_element_type=jnp.float32)
        kk = jnp.dot(hh, wk_r[...], preferred_element_type=jnp.float32)
        vv = jnp.dot(hh, wv_r[...], preferred_element_type=jnp.float32)
        kv_o[...] = jnp.concatenate([kk, vv], axis=1)

    return pl.pallas_call(
        body,
        grid=(N // NB,),
        in_specs=[_rows(NB, din), _full((din, HID)), _full((din, HID)),
                  _full((din, HID))],
        out_specs=[_rows(NB, HID), _rows(NB, 2 * HID)],
        out_shape=[jax.ShapeDtypeStruct((N, HID), jnp.float32),
                   jax.ShapeDtypeStruct((N, 2 * HID), jnp.float32)],
    )(h, wq, wk, wv)


def _tc_attn_edge(qd, kvs, rf8, w1p, b1, w2p, b2, sel):
    inv = 1.0 / (HD ** 0.5)

    def body(qd_r, kv_r, rf_r, w1_r, b1_r, w2_r, b2_r, sel_r, ev_o, ex_o):
        i = pl.program_id(0)
        gid = i * EB + lax.broadcasted_iota(jnp.int32, (EB, 1), 0)
        rmask = (gid < E).astype(jnp.float32)
        rad = jnp.dot(jnp.tanh(jnp.dot(rf_r[...], w1_r[...],
                                       preferred_element_type=jnp.float32)
                               + b1_r[...]),
                      w2_r[...], preferred_element_type=jnp.float32) + b2_r[...]
        qk = qd_r[...] * kv_r[:, 0:HID]
        logits = jnp.dot(qk, sel_r[...],
                         preferred_element_type=jnp.float32) * rad * inv
        hcol = lax.broadcasted_iota(jnp.int32, (EB, 8), 1)
        ex = jnp.where(hcol < HEADS, jnp.exp(logits), 0.0) * rmask
        ex128 = jnp.dot(ex, sel_r[...].T, preferred_element_type=jnp.float32)
        ev_o[...] = ex128 * kv_r[:, HID:2 * HID]
        ex_o[...] = jnp.concatenate(
            [ex, rmask, jnp.zeros((EB, 119), jnp.float32)], axis=1)

    return pl.pallas_call(
        body,
        grid=(EPAD // EB,),
        in_specs=[_rows(EB, HID), _rows(EB, 2 * HID), _rows(EB, 8),
                  _full((8, 128)), _full((1, 128)), _full((128, 8)),
                  _full((1, 8)), _full((128, 8))],
        out_specs=[_rows(EB, 128), _rows(EB, 128)],
        out_shape=[jax.ShapeDtypeStruct((EPAD, 128), jnp.float32),
                   jax.ShapeDtypeStruct((EPAD, 128), jnp.float32)],
    )(qd, kvs, rf8, w1p, b1, w2p, b2, sel)


def _tc_attn_finish(sv0, sv1, sx0, sx1, h, wo, wres, g, be, selt):
    din = h.shape[1]

    def body(sv0_r, sv1_r, sx0_r, sx1_r, h_r, wo_r, wres_r, g_r, be_r,
             selt_r, o_r):
        den8 = (sx0_r[...] + sx1_r[...])[:, 0:8]
        den128 = jnp.dot(den8, selt_r[...],
                         preferred_element_type=jnp.float32) + 1e-9
        agg = (sv0_r[...] + sv1_r[...]) / den128
        out = (jnp.dot(agg, wo_r[...], preferred_element_type=jnp.float32)
               + jnp.dot(h_r[...], wres_r[...],
                         preferred_element_type=jnp.float32))
        mu = jnp.mean(out, axis=1, keepdims=True)
        ctr = out - mu
        var = jnp.mean(ctr * ctr, axis=1, keepdims=True)
        o_r[...] = ctr / jnp.sqrt(var + 1e-5) * g_r[...] + be_r[...]

    return pl.pallas_call(
        body,
        grid=(N // NB,),
        in_specs=[_rows(NB, 128), _rows(NB, 128), _rows(NB, 128),
                  _rows(NB, 128), _rows(NB, din),
                  _full((HID, HID)), _full((din, HID)), _full((1, HID)),
                  _full((1, HID)), _full((8, 128))],
        out_specs=_rows(NB, HID),
        out_shape=jax.ShapeDtypeStruct((N, HID), jnp.float32),
    )(sv0, sv1, sx0, sx1, h, wo, wres, g, be, selt)


def _tc_conv_tables(h, wr, wc):
    def body(h_r, wr_r, wc_r, o_r):
        hh = h_r[...]
        o_r[...] = jnp.concatenate(
            [jnp.dot(hh, wr_r[...], preferred_element_type=jnp.float32),
             jnp.dot(hh, wc_r[...], preferred_element_type=jnp.float32)],
            axis=1)

    return pl.pallas_call(
        body,
        grid=(N // NB,),
        in_specs=[_rows(NB, HID), _full((HID, HID)), _full((HID, HID))],
        out_specs=_rows(NB, 2 * HID),
        out_shape=jax.ShapeDtypeStruct((N, 2 * HID), jnp.float32),
    )(h, wr, wc)


def _tc_conv_edge(ts, rf8, w1r, b1r, w2r, b2r, w1c, b1c, w2c, b2c):
    def body(ts_r, rf_r, w1r_r, b1r_r, w2r_r, b2r_r,
             w1c_r, b1c_r, w2c_r, b2c_r, o1_r, o2_r):
        i = pl.program_id(0)
        gid = i * EB + lax.broadcasted_iota(jnp.int32, (EB, 1), 0)
        rmask = (gid < E).astype(jnp.float32)
        rf = rf_r[...]

        def radial(w1, b1, w2, b2):
            t = jnp.tanh(jnp.dot(rf, w1[...],
                                 preferred_element_type=jnp.float32) + b1[...])
            return (jnp.dot(t, w2[...], preferred_element_type=jnp.float32)
                    + b2[...])[:, 0:1]

        radr = radial(w1r_r, b1r_r, w2r_r, b2r_r)
        radc = radial(w1c_r, b1c_r, w2c_r, b2c_r)
        o1_r[...] = radr * ts_r[:, 0:HID] * rmask
        o2_r[...] = radc * ts_r[:, HID:2 * HID] * rmask

    return pl.pallas_call(
        body,
        grid=(EPAD // EB,),
        in_specs=[_rows(EB, 2 * HID), _rows(EB, 8),
                  _full((8, 128)), _full((1, 128)), _full((128, 8)),
                  _full((1, 8)),
                  _full((8, 128)), _full((1, 128)), _full((128, 8)),
                  _full((1, 8))],
        out_specs=[_rows(EB, 128), _rows(EB, 128)],
        out_shape=[jax.ShapeDtypeStruct((EPAD, 128), jnp.float32),
                   jax.ShapeDtypeStruct((EPAD, 128), jnp.float32)],
    )(ts, rf8, w1r, b1r, w2r, b2r, w1c, b1c, w2c, b2c)


def _tc_conv_finish(s10, s11, s20, s21, dg0, dg1, h, wsr, wsc):
    def body(s10_r, s11_r, s20_r, s21_r, dg0_r, dg1_r, h_r, wsr_r, wsc_r,
             mr_o, mc_o):
        i = pl.program_id(0)
        deg = jnp.maximum((dg0_r[...] + dg1_r[...])[:, 8:9], 1.0)
        hh = h_r[...]
        hr = (s10_r[...] + s11_r[...]) / deg + jnp.dot(
            hh, wsr_r[...], preferred_element_type=jnp.float32)
        hc = (s20_r[...] + s21_r[...]) / deg + jnp.dot(
            hh, wsc_r[...], preferred_element_type=jnp.float32)
        mr = jnp.max(hr, axis=0, keepdims=True)
        mc = jnp.max(hc, axis=0, keepdims=True)

        @pl.when(i == 0)
        def _():
            mr_o[...] = mr
            mc_o[...] = mc

        @pl.when(i > 0)
        def _():
            mr_o[...] = jnp.maximum(mr_o[...], mr)
            mc_o[...] = jnp.maximum(mc_o[...], mc)

    return pl.pallas_call(
        body,
        grid=(N // NB,),
        in_specs=[_rows(NB, 128), _rows(NB, 128), _rows(NB, 128),
                  _rows(NB, 128), _rows(NB, 128), _rows(NB, 128),
                  _rows(NB, HID),
                  _full((HID, HID)), _full((HID, HID))],
        out_specs=[pl.BlockSpec((1, HID), lambda i: (0, 0)),
                   pl.BlockSpec((1, HID), lambda i: (0, 0))],
        out_shape=[jax.ShapeDtypeStruct((1, HID), jnp.float32),
                   jax.ShapeDtypeStruct((1, HID), jnp.float32)],
    )(s10, s11, s20, s21, dg0, dg1, h, wsr, wsc)


def kernel(x, pos, edge_attr, params, z, edge_index):
    f32 = jnp.float32
    src = edge_index[0].astype(jnp.int32)
    dst = edge_index[1].astype(jnp.int32)
    srcp = jnp.pad(src, (0, EPAD - E))
    dstp = jnp.pad(dst, (0, EPAD - E))

    drange = jnp.arange(HID) // HD
    sel = (drange[:, None] == jnp.arange(8)[None, :]).astype(f32)
    selt = sel.T

    zeros128 = jnp.zeros((NPAD, 128), f32)

    posp = jnp.pad(pos.astype(f32), ((0, 0), (0, 125)))
    pg = _sc_gather(posp, jnp.concatenate([srcp, dstp]))
    ps, pd = pg[:EPAD], pg[EPAD:]
    ea8 = jnp.pad(edge_attr.astype(f32), ((0, EPAD - E), (0, 4)))
    rf8 = _tc_rf8(ps, pd, ea8)

    p = params
    embp = jnp.pad(p['emb'].astype(f32), ((0, 1), (0, 0)))
    h0, ez = _tc_h0(x.astype(f32), z.reshape(N, 1).astype(jnp.int32),
                    p['W_fc1'], p['b_fc1'].reshape(1, -1),
                    p['W_fc2'], p['b_fc2'].reshape(1, -1), embp)
    h = jnp.concatenate([h0, ez], axis=1)

    sx_deg = None
    for lp in p['layers']:
        q, kv = _tc_qkv(h, lp['Wq'], lp['Wk'], lp['Wv'])
        qd = _sc_gather(q, dstp)
        kvs = _sc_gather(kv, srcp)
        w1p = jnp.pad(lp['Wr1'], ((0, 3), (0, 0)))
        w2p = jnp.pad(lp['Wr2'], ((0, 0), (0, 4)))
        b2p = jnp.pad(lp['br2'], (0, 4)).reshape(1, 8)
        ev, exd = _tc_attn_edge(qd, kvs, rf8, w1p, lp['br1'].reshape(1, -1),
                                w2p, b2p, sel)
        sv = _sc_scatter_add(ev, dstp, zeros128)
        sx = _sc_scatter_add(exd, dstp, zeros128)
        if sx_deg is None:
            sx_deg = sx
        h = _tc_attn_finish(sv[0, :N], sv[1, :N], sx[0, :N], sx[1, :N],
                            h, lp['Wo'], lp['Wres'],
                            lp['g'].reshape(1, -1), lp['be'].reshape(1, -1),
                            selt)

    cr, cc = p['reg'], p['cls']
    t = _tc_conv_tables(h, cr['W'], cc['W'])
    ts = _sc_gather(t, srcp)

    def rpad(cp):
        return (jnp.pad(cp['Wr1'], ((0, 3), (0, 0))),
                cp['br1'].reshape(1, -1),
                jnp.pad(cp['Wr2'], ((0, 0), (0, 7))),
                jnp.pad(cp['br2'], (0, 7)).reshape(1, 8))

    w1r, b1r, w2r, b2r = rpad(cr)
    w1c, b1c, w2c, b2c = rpad(cc)
    o1, o2 = _tc_conv_edge(ts, rf8, w1r, b1r, w2r, b2r, w1c, b1c, w2c, b2c)
    s1 = _sc_scatter_add(o1, dstp, zeros128)
    s2 = _sc_scatter_add(o2, dstp, zeros128)
    mr, mc = _tc_conv_finish(s1[0, :N], s1[1, :N], s2[0, :N], s2[1, :N],
                             sx_deg[0, :N], sx_deg[1, :N],
                             h, cr['Wself'], cc['Wself'])
    return (mr.reshape(HID), mc.reshape(HID))


# speedup vs baseline: 2.2316x; 1.1648x over previous
import functools

import jax
import jax.numpy as jnp
from jax import lax
from jax.experimental import pallas as pl
from jax.experimental.pallas import tpu as pltpu
from jax.experimental.pallas import tpu_sc as plsc

N = 10000
E = 160000
HID = 128
HEADS = 4
HD = 32

NC = 2
NS = 16
NW = NC * NS
CH = 128
EPAD = 163840
NPAD = 10240

EB = 512
NB = 400


def _sc_gather(table, idx):
    bp = idx.shape[0]
    d = table.shape[1]
    bw = bp // NW
    nit = bw // CH
    mesh = plsc.VectorSubcoreMesh(core_axis_name="c", subcore_axis_name="s")

    @functools.partial(
        pl.kernel, mesh=mesh,
        out_type=jax.ShapeDtypeStruct((bp, d), jnp.float32),
        scratch_types=[
            pltpu.VMEM((2, CH), jnp.int32),
            pltpu.VMEM((2, CH, d), jnp.float32),
            pltpu.SemaphoreType.DMA,
            pltpu.SemaphoreType.DMA,
        ],
    )
    def k(table_hbm, idx_hbm, out_hbm, idx_v, rows_v, sem0, sem1):
        cid = lax.axis_index("c")
        sid = lax.axis_index("s")
        base = (sid * NC + cid) * bw

        def fetch(j, b):
            pltpu.sync_copy(idx_hbm.at[pl.ds(base + j * CH, CH)],
                            idx_v.at[b])

            @pl.when(b == 0)
            def _():
                pltpu.async_copy(table_hbm.at[idx_v.at[b]], rows_v.at[b],
                                 sem0)

            @pl.when(b == 1)
            def _():
                pltpu.async_copy(table_hbm.at[idx_v.at[b]], rows_v.at[b],
                                 sem1)

        def drain(j, b):
            @pl.when(b == 0)
            def _():
                pltpu.make_async_copy(table_hbm.at[idx_v.at[b]],
                                      rows_v.at[b], sem0).wait()

            @pl.when(b == 1)
            def _():
                pltpu.make_async_copy(table_hbm.at[idx_v.at[b]],
                                      rows_v.at[b], sem1).wait()

            pltpu.sync_copy(rows_v.at[b], out_hbm.at[pl.ds(base + j * CH, CH)])

        fetch(0, 0)

        def body(j, c):
            b = lax.rem(j, 2)

            @pl.when(j + 1 < nit)
            def _():
                fetch(j + 1, 1 - b)

            drain(j, b)
            return c

        lax.fori_loop(0, nit, body, 0)

    return k(table, idx)


def _sc_scatter_add(vals, idx, zeros):
    d = vals.shape[1]
    bw = EPAD // NW
    nit = bw // CH
    rps = NPAD // NS
    mesh = plsc.VectorSubcoreMesh(core_axis_name="c", subcore_axis_name="s")

    @functools.partial(
        pl.kernel, mesh=mesh,
        out_type=jax.ShapeDtypeStruct((NC, NPAD, d), jnp.float32),
        scratch_types=[
            pltpu.VMEM((2, CH), jnp.int32),
            pltpu.VMEM((2, CH, d), jnp.float32),
            pltpu.VMEM_SHARED((NPAD, d), jnp.float32),
            pltpu.SemaphoreType.DMA,
            pltpu.SemaphoreType.DMA,
        ],
    )
    def k(vals_hbm, idx_hbm, zeros_hbm, out_hbm, idx_v, val_v, acc_sh,
          sem0, sem1):
        cid = lax.axis_index("c")
        sid = lax.axis_index("s")

        @pl.when(sid == 0)
        def _():
            pltpu.sync_copy(zeros_hbm, acc_sh)

        base = (sid * NC + cid) * bw

        def fetch(j, b):
            pltpu.sync_copy(idx_hbm.at[pl.ds(base + j * CH, CH)],
                            idx_v.at[b])

            @pl.when(b == 0)
            def _():
                pltpu.async_copy(vals_hbm.at[pl.ds(base + j * CH, CH)],
                                 val_v.at[b], sem0)

            @pl.when(b == 1)
            def _():
                pltpu.async_copy(vals_hbm.at[pl.ds(base + j * CH, CH)],
                                 val_v.at[b], sem1)

        def drain(j, b):
            @pl.when(b == 0)
            def _():
                pltpu.make_async_copy(vals_hbm.at[pl.ds(base + j * CH, CH)],
                                      val_v.at[b], sem0).wait()

            @pl.when(b == 1)
            def _():
                pltpu.make_async_copy(vals_hbm.at[pl.ds(base + j * CH, CH)],
                                      val_v.at[b], sem1).wait()

            pltpu.sync_copy(val_v.at[b], acc_sh.at[idx_v.at[b]], add=True)

        fetch(0, 0)
        plsc.subcore_barrier()

        def body(j, c):
            b = lax.rem(j, 2)

            @pl.when(j + 1 < nit)
            def _():
                fetch(j + 1, 1 - b)

            drain(j, b)
            return c

        lax.fori_loop(0, nit, body, 0)
        plsc.subcore_barrier()
        pltpu.sync_copy(acc_sh.at[pl.ds(sid * rps, rps)],
                        out_hbm.at[cid, pl.ds(sid * rps, rps)])

    return k(vals, idx, zeros)


def _full(shape):
    return pl.BlockSpec(shape, lambda i: tuple(0 for _ in shape))


def _rows(b, w):
    return pl.BlockSpec((b, w), lambda i: (i, 0))


def _tc_h0(x, z2, w1, b1, w2, b2, embp):
    def body(x_r, z_r, w1_r, b1_r, w2_r, b2_r, emb_r, h0_o, ez_o):
        a = jnp.maximum(jnp.dot(x_r[...], w1_r[...],
                                preferred_element_type=jnp.float32)
                        + b1_r[...], 0.0)
        h0_o[...] = jnp.dot(a, w2_r[...],
                            preferred_element_type=jnp.float32) + b2_r[...]
        zc = z_r[...]
        oh = (zc == lax.broadcasted_iota(jnp.int32, (NB, 96), 1)
              ).astype(jnp.float32)
        ez_o[...] = jnp.dot(oh, emb_r[...],
                            preferred_element_type=jnp.float32)

    return pl.pallas_call(
        body,
        grid=(N // NB,),
        in_specs=[_rows(NB, 128), _rows(NB, 1), _full((128, 256)),
                  _full((1, 256)), _full((256, 256)), _full((1, 256)),
                  _full((96, 32))],
        out_specs=[_rows(NB, 256), _rows(NB, 32)],
        out_shape=[jax.ShapeDtypeStruct((N, 256), jnp.float32),
                   jax.ShapeDtypeStruct((N, 32), jnp.float32)],
    )(x, z2, w1, b1, w2, b2, embp)


def _tc_rf8(ps, pd, ea8):
    def body(ps_r, pd_r, ea_r, o_r):
        dlt = pd_r[:, 0:3] - ps_r[:, 0:3]
        r = jnp.sqrt(jnp.sum(dlt * dlt, axis=1, keepdims=True) + 1e-8)
        o_r[...] = jnp.concatenate(
            [r, ea_r[:, 0:4], jnp.zeros((EB, 3), jnp.float32)], axis=1)

    return pl.pallas_call(
        body,
        grid=(EPAD // EB,),
        in_specs=[_rows(EB, 128), _rows(EB, 128), _rows(EB, 8)],
        out_specs=_rows(EB, 8),
        out_shape=jax.ShapeDtypeStruct((EPAD, 8), jnp.float32),
    )(ps, pd, ea8)


def _tc_qkv(h, wq, wk, wv):
    din = h.shape[1]

    def body(h_r, wq_r, wk_r, wv_r, q_o, kv_o):
        hh = h_r[...]
        q_o[...] = jnp.dot(hh, wq_r[...], preferred_element_type=jnp.float32)
        kk = jnp.dot(hh, wk_r[...], preferred_element_type=jnp.float32)
        vv = jnp.dot(hh, wv_r[...], preferred_element_type=jnp.float32)
        kv_o[...] = jnp.concatenate([kk, vv], axis=1)

    return pl.pallas_call(
        body,
        grid=(N // NB,),
        in_specs=[_rows(NB, din), _full((din, HID)), _full((din, HID)),
                  _full((din, HID))],
        out_specs=[_rows(NB, HID), _rows(NB, 2 * HID)],
        out_shape=[jax.ShapeDtypeStruct((N, HID), jnp.float32),
                   jax.ShapeDtypeStruct((N, 2 * HID), jnp.float32)],
    )(h, wq, wk, wv)


def _tc_attn_edge(qd, kvs, rf8, w1p, b1, w2p, b2, sel):
    inv = 1.0 / (HD ** 0.5)

    def body(qd_r, kv_r, rf_r, w1_r, b1_r, w2_r, b2_r, sel_r, ev_o, ex_o):
        i = pl.program_id(0)
        gid = i * EB + lax.broadcasted_iota(jnp.int32, (EB, 1), 0)
        rmask = (gid < E).astype(jnp.float32)
        rad = jnp.dot(jnp.tanh(jnp.dot(rf_r[...], w1_r[...],
                                       preferred_element_type=jnp.float32)
                               + b1_r[...]),
                      w2_r[...], preferred_element_type=jnp.float32) + b2_r[...]
        qk = qd_r[...] * kv_r[:, 0:HID]
        logits = jnp.dot(qk, sel_r[...],
                         preferred_element_type=jnp.float32) * rad * inv
        hcol = lax.broadcasted_iota(jnp.int32, (EB, 8), 1)
        ex = jnp.where(hcol < HEADS, jnp.exp(logits), 0.0) * rmask
        ex128 = jnp.dot(ex, sel_r[...].T, preferred_element_type=jnp.float32)
        ev_o[...] = ex128 * kv_r[:, HID:2 * HID]
        ex_o[...] = jnp.concatenate(
            [ex, rmask, jnp.zeros((EB, 119), jnp.float32)], axis=1)

    return pl.pallas_call(
        body,
        grid=(EPAD // EB,),
        in_specs=[_rows(EB, HID), _rows(EB, 2 * HID), _rows(EB, 8),
                  _full((8, 128)), _full((1, 128)), _full((128, 8)),
                  _full((1, 8)), _full((128, 8))],
        out_specs=[_rows(EB, 128), _rows(EB, 128)],
        out_shape=[jax.ShapeDtypeStruct((EPAD, 128), jnp.float32),
                   jax.ShapeDtypeStruct((EPAD, 128), jnp.float32)],
    )(qd, kvs, rf8, w1p, b1, w2p, b2, sel)


def _tc_attn_finish(sv0, sv1, sx0, sx1, h, wo, wres, g, be, selt):
    din = h.shape[1]

    def body(sv0_r, sv1_r, sx0_r, sx1_r, h_r, wo_r, wres_r, g_r, be_r,
             selt_r, o_r):
        den8 = (sx0_r[...] + sx1_r[...])[:, 0:8]
        den128 = jnp.dot(den8, selt_r[...],
                         preferred_element_type=jnp.float32) + 1e-9
        agg = (sv0_r[...] + sv1_r[...]) / den128
        out = (jnp.dot(agg, wo_r[...], preferred_element_type=jnp.float32)
               + jnp.dot(h_r[...], wres_r[...],
                         preferred_element_type=jnp.float32))
        mu = jnp.mean(out, axis=1, keepdims=True)
        ctr = out - mu
        var = jnp.mean(ctr * ctr, axis=1, keepdims=True)
        o_r[...] = ctr / jnp.sqrt(var + 1e-5) * g_r[...] + be_r[...]

    return pl.pallas_call(
        body,
        grid=(N // NB,),
        in_specs=[_rows(NB, 128), _rows(NB, 128), _rows(NB, 128),
                  _rows(NB, 128), _rows(NB, din),
                  _full((HID, HID)), _full((din, HID)), _full((1, HID)),
                  _full((1, HID)), _full((8, 128))],
        out_specs=_rows(NB, HID),
        out_shape=jax.ShapeDtypeStruct((N, HID), jnp.float32),
    )(sv0, sv1, sx0, sx1, h, wo, wres, g, be, selt)


def _tc_conv_tables(h, wr, wc):
    def body(h_r, wr_r, wc_r, o_r):
        hh = h_r[...]
        o_r[...] = jnp.concatenate(
            [jnp.dot(hh, wr_r[...], preferred_element_type=jnp.float32),
             jnp.dot(hh, wc_r[...], preferred_element_type=jnp.float32)],
            axis=1)

    return pl.pallas_call(
        body,
        grid=(N // NB,),
        in_specs=[_rows(NB, HID), _full((HID, HID)), _full((HID, HID))],
        out_specs=_rows(NB, 2 * HID),
        out_shape=jax.ShapeDtypeStruct((N, 2 * HID), jnp.float32),
    )(h, wr, wc)


def _tc_conv_edge(ts, rf8, w1r, b1r, w2r, b2r, w1c, b1c, w2c, b2c):
    def body(ts_r, rf_r, w1r_r, b1r_r, w2r_r, b2r_r,
             w1c_r, b1c_r, w2c_r, b2c_r, o1_r, o2_r):
        i = pl.program_id(0)
        gid = i * EB + lax.broadcasted_iota(jnp.int32, (EB, 1), 0)
        rmask = (gid < E).astype(jnp.float32)
        rf = rf_r[...]

        def radial(w1, b1, w2, b2):
            t = jnp.tanh(jnp.dot(rf, w1[...],
                                 preferred_element_type=jnp.float32) + b1[...])
            return (jnp.dot(t, w2[...], preferred_element_type=jnp.float32)
                    + b2[...])[:, 0:1]

        radr = radial(w1r_r, b1r_r, w2r_r, b2r_r)
        radc = radial(w1c_r, b1c_r, w2c_r, b2c_r)
        o1_r[...] = radr * ts_r[:, 0:HID] * rmask
        o2_r[...] = radc * ts_r[:, HID:2 * HID] * rmask

    return pl.pallas_call(
        body,
        grid=(EPAD // EB,),
        in_specs=[_rows(EB, 2 * HID), _rows(EB, 8),
                  _full((8, 128)), _full((1, 128)), _full((128, 8)),
                  _full((1, 8)),
                  _full((8, 128)), _full((1, 128)), _full((128, 8)),
                  _full((1, 8))],
        out_specs=[_rows(EB, 128), _rows(EB, 128)],
        out_shape=[jax.ShapeDtypeStruct((EPAD, 128), jnp.float32),
                   jax.ShapeDtypeStruct((EPAD, 128), jnp.float32)],
    )(ts, rf8, w1r, b1r, w2r, b2r, w1c, b1c, w2c, b2c)


def _tc_conv_finish(s10, s11, s20, s21, dg0, dg1, h, wsr, wsc):
    def body(s10_r, s11_r, s20_r, s21_r, dg0_r, dg1_r, h_r, wsr_r, wsc_r,
             mr_o, mc_o):
        i = pl.program_id(0)
        deg = jnp.maximum((dg0_r[...] + dg1_r[...])[:, 8:9], 1.0)
        hh = h_r[...]
        hr = (s10_r[...] + s11_r[...]) / deg + jnp.dot(
            hh, wsr_r[...], preferred_element_type=jnp.float32)
        hc = (s20_r[...] + s21_r[...]) / deg + jnp.dot(
            hh, wsc_r[...], preferred_element_type=jnp.float32)
        mr = jnp.max(hr, axis=0, keepdims=True)
        mc = jnp.max(hc, axis=0, keepdims=True)

        @pl.when(i == 0)
        def _():
            mr_o[...] = mr
            mc_o[...] = mc

        @pl.when(i > 0)
        def _():
            mr_o[...] = jnp.maximum(mr_o[...], mr)
            mc_o[...] = jnp.maximum(mc_o[...], mc)

    return pl.pallas_call(
        body,
        grid=(N // NB,),
        in_specs=[_rows(NB, 128), _rows(NB, 128), _rows(NB, 128),
                  _rows(NB, 128), _rows(NB, 128), _rows(NB, 128),
                  _rows(NB, HID),
                  _full((HID, HID)), _full((HID, HID))],
        out_specs=[pl.BlockSpec((1, HID), lambda i: (0, 0)),
                   pl.BlockSpec((1, HID), lambda i: (0, 0))],
        out_shape=[jax.ShapeDtypeStruct((1, HID), jnp.float32),
                   jax.ShapeDtypeStruct((1, HID), jnp.float32)],
    )(s10, s11, s20, s21, dg0, dg1, h, wsr, wsc)


def kernel(x, pos, edge_attr, params, z, edge_index):
    f32 = jnp.float32
    src = edge_index[0].astype(jnp.int32)
    dst = edge_index[1].astype(jnp.int32)
    srcp = jnp.pad(src, (0, EPAD - E))
    dstp = jnp.pad(dst, (0, EPAD - E))

    drange = jnp.arange(HID) // HD
    sel = (drange[:, None] == jnp.arange(8)[None, :]).astype(f32)
    selt = sel.T

    zeros128 = jnp.zeros((NPAD, 128), f32)

    posp = jnp.pad(pos.astype(f32), ((0, 0), (0, 125)))
    pg = _sc_gather(posp, jnp.concatenate([srcp, dstp]))
    ps, pd = pg[:EPAD], pg[EPAD:]
    ea8 = jnp.pad(edge_attr.astype(f32), ((0, EPAD - E), (0, 4)))
    rf8 = _tc_rf8(ps, pd, ea8)

    p = params
    embp = jnp.pad(p['emb'].astype(f32), ((0, 1), (0, 0)))
    h0, ez = _tc_h0(x.astype(f32), z.reshape(N, 1).astype(jnp.int32),
                    p['W_fc1'], p['b_fc1'].reshape(1, -1),
                    p['W_fc2'], p['b_fc2'].reshape(1, -1), embp)
    h = jnp.concatenate([h0, ez], axis=1)

    sx_deg = None
    for lp in p['layers']:
        q, kv = _tc_qkv(h, lp['Wq'], lp['Wk'], lp['Wv'])
        qd = _sc_gather(q, dstp)
        kvs = _sc_gather(kv, srcp)
        w1p = jnp.pad(lp['Wr1'], ((0, 3), (0, 0)))
        w2p = jnp.pad(lp['Wr2'], ((0, 0), (0, 4)))
        b2p = jnp.pad(lp['br2'], (0, 4)).reshape(1, 8)
        ev, exd = _tc_attn_edge(qd, kvs, rf8, w1p, lp['br1'].reshape(1, -1),
                                w2p, b2p, sel)
        sv = _sc_scatter_add(ev, dstp, zeros128)
        sx = _sc_scatter_add(exd, dstp, zeros128)
        if sx_deg is None:
            sx_deg = sx
        h = _tc_attn_finish(sv[0, :N], sv[1, :N], sx[0, :N], sx[1, :N],
                            h, lp['Wo'], lp['Wres'],
                            lp['g'].reshape(1, -1), lp['be'].reshape(1, -1),
                            selt)

    cr, cc = p['reg'], p['cls']
    t = _tc_conv_tables(h, cr['W'], cc['W'])
    ts = _sc_gather(t, srcp)

    def rpad(cp):
        return (jnp.pad(cp['Wr1'], ((0, 3), (0, 0))),
                cp['br1'].reshape(1, -1),
                jnp.pad(cp['Wr2'], ((0, 0), (0, 7))),
                jnp.pad(cp['br2'], (0, 7)).reshape(1, 8))

    w1r, b1r, w2r, b2r = rpad(cr)
    w1c, b1c, w2c, b2c = rpad(cc)
    o1, o2 = _tc_conv_edge(ts, rf8, w1r, b1r, w2r, b2r, w1c, b1c, w2c, b2c)
    s1 = _sc_scatter_add(o1, dstp, zeros128)
    s2 = _sc_scatter_add(o2, dstp, zeros128)
    mr, mc = _tc_conv_finish(s1[0, :N], s1[1, :N], s2[0, :N], s2[1, :N],
                             sx_deg[0, :N], sx_deg[1, :N],
                             h, cr['Wself'], cc['Wself'])
    return (mr.reshape(HID), mc.reshape(HID))


# speedup vs baseline: 2.2882x; 1.0253x over previous
import functools

import jax
import jax.numpy as jnp
from jax import lax
from jax.experimental import pallas as pl
from jax.experimental.pallas import tpu as pltpu
from jax.experimental.pallas import tpu_sc as plsc

N = 10000
E = 160000
HID = 128
HEADS = 4
HD = 32

NC = 2
NS = 16
NW = NC * NS
CH = 128
EPAD = 163840
NPAD = 10240

EB = 512
NB = 400


def _sc_gather(table, idx):
    bp = idx.shape[0]
    d = table.shape[1]
    bw = bp // NW
    nit = bw // CH
    mesh = plsc.VectorSubcoreMesh(core_axis_name="c", subcore_axis_name="s")

    @functools.partial(
        pl.kernel, mesh=mesh,
        out_type=jax.ShapeDtypeStruct((bp, d), jnp.float32),
        scratch_types=[
            pltpu.VMEM((2, CH), jnp.int32),
            pltpu.VMEM((2, CH, d), jnp.float32),
            pltpu.SemaphoreType.DMA,
            pltpu.SemaphoreType.DMA,
        ],
    )
    def k(table_hbm, idx_hbm, out_hbm, idx_v, rows_v, sem0, sem1):
        cid = lax.axis_index("c")
        sid = lax.axis_index("s")
        base = (sid * NC + cid) * bw

        def fetch(j, b):
            pltpu.sync_copy(idx_hbm.at[pl.ds(base + j * CH, CH)],
                            idx_v.at[b])

            @pl.when(b == 0)
            def _():
                pltpu.async_copy(table_hbm.at[idx_v.at[b]], rows_v.at[b],
                                 sem0)

            @pl.when(b == 1)
            def _():
                pltpu.async_copy(table_hbm.at[idx_v.at[b]], rows_v.at[b],
                                 sem1)

        def drain(j, b):
            @pl.when(b == 0)
            def _():
                pltpu.make_async_copy(table_hbm.at[idx_v.at[b]],
                                      rows_v.at[b], sem0).wait()

            @pl.when(b == 1)
            def _():
                pltpu.make_async_copy(table_hbm.at[idx_v.at[b]],
                                      rows_v.at[b], sem1).wait()

            pltpu.sync_copy(rows_v.at[b], out_hbm.at[pl.ds(base + j * CH, CH)])

        fetch(0, 0)

        def body(j, c):
            b = lax.rem(j, 2)

            @pl.when(j + 1 < nit)
            def _():
                fetch(j + 1, 1 - b)

            drain(j, b)
            return c

        lax.fori_loop(0, nit, body, 0)

    return k(table, idx)


def _sc_scatter_add(vals, idx, zeros):
    d = vals.shape[1]
    bw = EPAD // NW
    nit = bw // CH
    rps = NPAD // NS
    mesh = plsc.VectorSubcoreMesh(core_axis_name="c", subcore_axis_name="s")

    @functools.partial(
        pl.kernel, mesh=mesh,
        out_type=jax.ShapeDtypeStruct((NC, NPAD, d), jnp.float32),
        scratch_types=[
            pltpu.VMEM((2, CH), jnp.int32),
            pltpu.VMEM((2, CH, d), jnp.float32),
            pltpu.VMEM_SHARED((NPAD, d), jnp.float32),
            pltpu.SemaphoreType.DMA,
            pltpu.SemaphoreType.DMA,
        ],
    )
    def k(vals_hbm, idx_hbm, zeros_hbm, out_hbm, idx_v, val_v, acc_sh,
          sem0, sem1):
        cid = lax.axis_index("c")
        sid = lax.axis_index("s")

        @pl.when(sid == 0)
        def _():
            pltpu.sync_copy(zeros_hbm, acc_sh)

        base = (sid * NC + cid) * bw

        def fetch(j, b):
            pltpu.sync_copy(idx_hbm.at[pl.ds(base + j * CH, CH)],
                            idx_v.at[b])

            @pl.when(b == 0)
            def _():
                pltpu.async_copy(vals_hbm.at[pl.ds(base + j * CH, CH)],
                                 val_v.at[b], sem0)

            @pl.when(b == 1)
            def _():
                pltpu.async_copy(vals_hbm.at[pl.ds(base + j * CH, CH)],
                                 val_v.at[b], sem1)

        def drain(j, b):
            @pl.when(b == 0)
            def _():
                pltpu.make_async_copy(vals_hbm.at[pl.ds(base + j * CH, CH)],
                                      val_v.at[b], sem0).wait()

            @pl.when(b == 1)
            def _():
                pltpu.make_async_copy(vals_hbm.at[pl.ds(base + j * CH, CH)],
                                      val_v.at[b], sem1).wait()

            pltpu.sync_copy(val_v.at[b], acc_sh.at[idx_v.at[b]], add=True)

        fetch(0, 0)
        plsc.subcore_barrier()

        def body(j, c):
            b = lax.rem(j, 2)

            @pl.when(j + 1 < nit)
            def _():
                fetch(j + 1, 1 - b)

            drain(j, b)
            return c

        lax.fori_loop(0, nit, body, 0)
        plsc.subcore_barrier()
        pltpu.sync_copy(acc_sh.at[pl.ds(sid * rps, rps)],
                        out_hbm.at[cid, pl.ds(sid * rps, rps)])

    return k(vals, idx, zeros)


def _full(shape):
    return pl.BlockSpec(shape, lambda i: tuple(0 for _ in shape))


def _rows(b, w):
    return pl.BlockSpec((b, w), lambda i: (i, 0))


def _tc_h0(x, z2, w1, b1, w2, b2, embp):
    def body(x_r, z_r, w1_r, b1_r, w2_r, b2_r, emb_r, h0_o, ez_o):
        a = jnp.maximum(jnp.dot(x_r[...], w1_r[...],
                                preferred_element_type=jnp.float32)
                        + b1_r[...], 0.0)
        h0_o[...] = jnp.dot(a, w2_r[...],
                            preferred_element_type=jnp.float32) + b2_r[...]
        zc = z_r[...]
        oh = (zc == lax.broadcasted_iota(jnp.int32, (NB, 96), 1)
              ).astype(jnp.float32)
        ez_o[...] = jnp.dot(oh, emb_r[...],
                            preferred_element_type=jnp.float32)

    return pl.pallas_call(
        body,
        grid=(N // NB,),
        in_specs=[_rows(NB, 128), _rows(NB, 1), _full((128, 256)),
                  _full((1, 256)), _full((256, 256)), _full((1, 256)),
                  _full((96, 32))],
        out_specs=[_rows(NB, 256), _rows(NB, 32)],
        out_shape=[jax.ShapeDtypeStruct((N, 256), jnp.float32),
                   jax.ShapeDtypeStruct((N, 32), jnp.float32)],
    )(x, z2, w1, b1, w2, b2, embp)


def _tc_rf8(ps, pd, ea8):
    def body(ps_r, pd_r, ea_r, o_r):
        dlt = pd_r[:, 0:3] - ps_r[:, 0:3]
        r = jnp.sqrt(jnp.sum(dlt * dlt, axis=1, keepdims=True) + 1e-8)
        o_r[...] = jnp.concatenate(
            [r, ea_r[:, 0:4], jnp.zeros((EB, 3), jnp.float32)], axis=1)

    return pl.pallas_call(
        body,
        grid=(EPAD // EB,),
        in_specs=[pl.BlockSpec((EB, 128), lambda i: (i, 2)),
                  pl.BlockSpec((EB, 128), lambda i: (i, 1)),
                  _rows(EB, 8)],
        out_specs=_rows(EB, 8),
        out_shape=jax.ShapeDtypeStruct((EPAD, 8), jnp.float32),
    )(ps, pd, ea8)


def _tc_qkv0(h, wq, wk, wv, posp):
    din = h.shape[1]

    def body(h_r, wq_r, wk_r, wv_r, pp_r, q_o, kv_o):
        hh = h_r[...]
        pp = pp_r[...]
        qq = jnp.dot(hh, wq_r[...], preferred_element_type=jnp.float32)
        kk = jnp.dot(hh, wk_r[...], preferred_element_type=jnp.float32)
        vv = jnp.dot(hh, wv_r[...], preferred_element_type=jnp.float32)
        q_o[...] = jnp.concatenate([qq, pp], axis=1)
        kv_o[...] = jnp.concatenate([kk, vv, pp], axis=1)

    return pl.pallas_call(
        body,
        grid=(N // NB,),
        in_specs=[_rows(NB, din), _full((din, HID)), _full((din, HID)),
                  _full((din, HID)), _rows(NB, 128)],
        out_specs=[_rows(NB, 2 * HID), _rows(NB, 3 * HID)],
        out_shape=[jax.ShapeDtypeStruct((N, 2 * HID), jnp.float32),
                   jax.ShapeDtypeStruct((N, 3 * HID), jnp.float32)],
    )(h, wq, wk, wv, posp)


def _tc_qkv(h, wq, wk, wv):
    din = h.shape[1]

    def body(h_r, wq_r, wk_r, wv_r, q_o, kv_o):
        hh = h_r[...]
        q_o[...] = jnp.dot(hh, wq_r[...], preferred_element_type=jnp.float32)
        kk = jnp.dot(hh, wk_r[...], preferred_element_type=jnp.float32)
        vv = jnp.dot(hh, wv_r[...], preferred_element_type=jnp.float32)
        kv_o[...] = jnp.concatenate([kk, vv], axis=1)

    return pl.pallas_call(
        body,
        grid=(N // NB,),
        in_specs=[_rows(NB, din), _full((din, HID)), _full((din, HID)),
                  _full((din, HID))],
        out_specs=[_rows(NB, HID), _rows(NB, 2 * HID)],
        out_shape=[jax.ShapeDtypeStruct((N, HID), jnp.float32),
                   jax.ShapeDtypeStruct((N, 2 * HID), jnp.float32)],
    )(h, wq, wk, wv)


def _tc_attn_edge(qd, kvs, rf8, w1p, b1, w2p, b2, sel):
    inv = 1.0 / (HD ** 0.5)

    def body(qd_r, kv_r, rf_r, w1_r, b1_r, w2_r, b2_r, sel_r, ev_o, ex_o):
        i = pl.program_id(0)
        gid = i * EB + lax.broadcasted_iota(jnp.int32, (EB, 1), 0)
        rmask = (gid < E).astype(jnp.float32)
        rad = jnp.dot(jnp.tanh(jnp.dot(rf_r[...], w1_r[...],
                                       preferred_element_type=jnp.float32)
                               + b1_r[...]),
                      w2_r[...], preferred_element_type=jnp.float32) + b2_r[...]
        qk = qd_r[:, 0:HID] * kv_r[:, 0:HID]
        logits = jnp.dot(qk, sel_r[...],
                         preferred_element_type=jnp.float32) * rad * inv
        hcol = lax.broadcasted_iota(jnp.int32, (EB, 8), 1)
        ex = jnp.where(hcol < HEADS, jnp.exp(logits), 0.0) * rmask
        ex128 = jnp.dot(ex, sel_r[...].T, preferred_element_type=jnp.float32)
        ev_o[...] = ex128 * kv_r[:, HID:2 * HID]
        ex_o[...] = jnp.concatenate(
            [ex, rmask, jnp.zeros((EB, 119), jnp.float32)], axis=1)

    return pl.pallas_call(
        body,
        grid=(EPAD // EB,),
        in_specs=[_rows(EB, qd.shape[1]), _rows(EB, kvs.shape[1]),
                  _rows(EB, 8),
                  _full((8, 128)), _full((1, 128)), _full((128, 8)),
                  _full((1, 8)), _full((128, 8))],
        out_specs=[_rows(EB, 128), _rows(EB, 128)],
        out_shape=[jax.ShapeDtypeStruct((EPAD, 128), jnp.float32),
                   jax.ShapeDtypeStruct((EPAD, 128), jnp.float32)],
    )(qd, kvs, rf8, w1p, b1, w2p, b2, sel)


def _tc_attn_finish(sv0, sv1, sx0, sx1, h, wo, wres, g, be, selt):
    din = h.shape[1]

    def body(sv0_r, sv1_r, sx0_r, sx1_r, h_r, wo_r, wres_r, g_r, be_r,
             selt_r, o_r):
        den8 = (sx0_r[...] + sx1_r[...])[:, 0:8]
        den128 = jnp.dot(den8, selt_r[...],
                         preferred_element_type=jnp.float32) + 1e-9
        agg = (sv0_r[...] + sv1_r[...]) / den128
        out = (jnp.dot(agg, wo_r[...], preferred_element_type=jnp.float32)
               + jnp.dot(h_r[...], wres_r[...],
                         preferred_element_type=jnp.float32))
        mu = jnp.mean(out, axis=1, keepdims=True)
        ctr = out - mu
        var = jnp.mean(ctr * ctr, axis=1, keepdims=True)
        o_r[...] = ctr / jnp.sqrt(var + 1e-5) * g_r[...] + be_r[...]

    return pl.pallas_call(
        body,
        grid=(N // NB,),
        in_specs=[_rows(NB, 128), _rows(NB, 128), _rows(NB, 128),
                  _rows(NB, 128), _rows(NB, din),
                  _full((HID, HID)), _full((din, HID)), _full((1, HID)),
                  _full((1, HID)), _full((8, 128))],
        out_specs=_rows(NB, HID),
        out_shape=jax.ShapeDtypeStruct((N, HID), jnp.float32),
    )(sv0, sv1, sx0, sx1, h, wo, wres, g, be, selt)


def _tc_conv_tables(h, wr, wc):
    def body(h_r, wr_r, wc_r, o_r):
        hh = h_r[...]
        o_r[...] = jnp.concatenate(
            [jnp.dot(hh, wr_r[...], preferred_element_type=jnp.float32),
             jnp.dot(hh, wc_r[...], preferred_element_type=jnp.float32)],
            axis=1)

    return pl.pallas_call(
        body,
        grid=(N // NB,),
        in_specs=[_rows(NB, HID), _full((HID, HID)), _full((HID, HID))],
        out_specs=_rows(NB, 2 * HID),
        out_shape=jax.ShapeDtypeStruct((N, 2 * HID), jnp.float32),
    )(h, wr, wc)


def _tc_conv_edge(ts, rf8, w1r, b1r, w2r, b2r, w1c, b1c, w2c, b2c):
    def body(ts_r, rf_r, w1r_r, b1r_r, w2r_r, b2r_r,
             w1c_r, b1c_r, w2c_r, b2c_r, o1_r, o2_r):
        i = pl.program_id(0)
        gid = i * EB + lax.broadcasted_iota(jnp.int32, (EB, 1), 0)
        rmask = (gid < E).astype(jnp.float32)
        rf = rf_r[...]

        def radial(w1, b1, w2, b2):
            t = jnp.tanh(jnp.dot(rf, w1[...],
                                 preferred_element_type=jnp.float32) + b1[...])
            return (jnp.dot(t, w2[...], preferred_element_type=jnp.float32)
                    + b2[...])[:, 0:1]

        radr = radial(w1r_r, b1r_r, w2r_r, b2r_r)
        radc = radial(w1c_r, b1c_r, w2c_r, b2c_r)
        o1_r[...] = radr * ts_r[:, 0:HID] * rmask
        o2_r[...] = radc * ts_r[:, HID:2 * HID] * rmask

    return pl.pallas_call(
        body,
        grid=(EPAD // EB,),
        in_specs=[_rows(EB, 2 * HID), _rows(EB, 8),
                  _full((8, 128)), _full((1, 128)), _full((128, 8)),
                  _full((1, 8)),
                  _full((8, 128)), _full((1, 128)), _full((128, 8)),
                  _full((1, 8))],
        out_specs=[_rows(EB, 128), _rows(EB, 128)],
        out_shape=[jax.ShapeDtypeStruct((EPAD, 128), jnp.float32),
                   jax.ShapeDtypeStruct((EPAD, 128), jnp.float32)],
    )(ts, rf8, w1r, b1r, w2r, b2r, w1c, b1c, w2c, b2c)


def _tc_conv_finish(s10, s11, s20, s21, dg0, dg1, h, wsr, wsc):
    def body(s10_r, s11_r, s20_r, s21_r, dg0_r, dg1_r, h_r, wsr_r, wsc_r,
             mr_o, mc_o):
        i = pl.program_id(0)
        deg = jnp.maximum((dg0_r[...] + dg1_r[...])[:, 8:9], 1.0)
        hh = h_r[...]
        hr = (s10_r[...] + s11_r[...]) / deg + jnp.dot(
            hh, wsr_r[...], preferred_element_type=jnp.float32)
        hc = (s20_r[...] + s21_r[...]) / deg + jnp.dot(
            hh, wsc_r[...], preferred_element_type=jnp.float32)
        mr = jnp.max(hr, axis=0, keepdims=True)
        mc = jnp.max(hc, axis=0, keepdims=True)

        @pl.when(i == 0)
        def _():
            mr_o[...] = mr
            mc_o[...] = mc

        @pl.when(i > 0)
        def _():
            mr_o[...] = jnp.maximum(mr_o[...], mr)
            mc_o[...] = jnp.maximum(mc_o[...], mc)

    return pl.pallas_call(
        body,
        grid=(N // NB,),
        in_specs=[_rows(NB, 128), _rows(NB, 128), _rows(NB, 128),
                  _rows(NB, 128), _rows(NB, 128), _rows(NB, 128),
                  _rows(NB, HID),
                  _full((HID, HID)), _full((HID, HID))],
        out_specs=[pl.BlockSpec((1, HID), lambda i: (0, 0)),
                   pl.BlockSpec((1, HID), lambda i: (0, 0))],
        out_shape=[jax.ShapeDtypeStruct((1, HID), jnp.float32),
                   jax.ShapeDtypeStruct((1, HID), jnp.float32)],
    )(s10, s11, s20, s21, dg0, dg1, h, wsr, wsc)


def kernel(x, pos, edge_attr, params, z, edge_index):
    f32 = jnp.float32
    src = edge_index[0].astype(jnp.int32)
    dst = edge_index[1].astype(jnp.int32)
    srcp = jnp.pad(src, (0, EPAD - E))
    dstp = jnp.pad(dst, (0, EPAD - E))

    drange = jnp.arange(HID) // HD
    sel = (drange[:, None] == jnp.arange(8)[None, :]).astype(f32)
    selt = sel.T

    zeros128 = jnp.zeros((NPAD, 128), f32)

    posp = jnp.pad(pos.astype(f32), ((0, 0), (0, 125)))
    ea8 = jnp.pad(edge_attr.astype(f32), ((0, EPAD - E), (0, 4)))

    p = params
    embp = jnp.pad(p['emb'].astype(f32), ((0, 1), (0, 0)))
    h0, ez = _tc_h0(x.astype(f32), z.reshape(N, 1).astype(jnp.int32),
                    p['W_fc1'], p['b_fc1'].reshape(1, -1),
                    p['W_fc2'], p['b_fc2'].reshape(1, -1), embp)
    h = jnp.concatenate([h0, ez], axis=1)

    sx_deg = None
    rf8 = None
    for li, lp in enumerate(p['layers']):
        if li == 0:
            q, kv = _tc_qkv0(h, lp['Wq'], lp['Wk'], lp['Wv'], posp)
        else:
            q, kv = _tc_qkv(h, lp['Wq'], lp['Wk'], lp['Wv'])
        qd = _sc_gather(q, dstp)
        kvs = _sc_gather(kv, srcp)
        if li == 0:
            rf8 = _tc_rf8(kvs, qd, ea8)
        w1p = jnp.pad(lp['Wr1'], ((0, 3), (0, 0)))
        w2p = jnp.pad(lp['Wr2'], ((0, 0), (0, 4)))
        b2p = jnp.pad(lp['br2'], (0, 4)).reshape(1, 8)
        ev, exd = _tc_attn_edge(qd, kvs, rf8, w1p, lp['br1'].reshape(1, -1),
                                w2p, b2p, sel)
        sv = _sc_scatter_add(ev, dstp, zeros128)
        sx = _sc_scatter_add(exd, dstp, zeros128)
        if sx_deg is None:
            sx_deg = sx
        h = _tc_attn_finish(sv[0, :N], sv[1, :N], sx[0, :N], sx[1, :N],
                            h, lp['Wo'], lp['Wres'],
                            lp['g'].reshape(1, -1), lp['be'].reshape(1, -1),
                            selt)

    cr, cc = p['reg'], p['cls']
    t = _tc_conv_tables(h, cr['W'], cc['W'])
    ts = _sc_gather(t, srcp)

    def rpad(cp):
        return (jnp.pad(cp['Wr1'], ((0, 3), (0, 0))),
                cp['br1'].reshape(1, -1),
                jnp.pad(cp['Wr2'], ((0, 0), (0, 7))),
                jnp.pad(cp['br2'], (0, 7)).reshape(1, 8))

    w1r, b1r, w2r, b2r = rpad(cr)
    w1c, b1c, w2c, b2c = rpad(cc)
    o1, o2 = _tc_conv_edge(ts, rf8, w1r, b1r, w2r, b2r, w1c, b1c, w2c, b2c)
    s1 = _sc_scatter_add(o1, dstp, zeros128)
    s2 = _sc_scatter_add(o2, dstp, zeros128)
    mr, mc = _tc_conv_finish(s1[0, :N], s1[1, :N], s2[0, :N], s2[1, :N],
                             sx_deg[0, :N], sx_deg[1, :N],
                             h, cr['Wself'], cc['Wself'])
    return (mr.reshape(HID), mc.reshape(HID))
